# baseline (speedup 1.0000x reference)
import jax
import jax.numpy as jnp
from jax.experimental import pallas as pl

_HIDDEN = 2048
_INTER = 2048
_TOKENS = 32
_BJ = 256
_Q = 4
_HQ = _HIDDEN // _Q
_NQ = _HIDDEN // _Q


def _mlp_kernel(x_ref, *refs):
    wg = refs[0:_Q]
    wu = refs[_Q:2 * _Q]
    wd = refs[2 * _Q:3 * _Q]
    o_ref = refs[3 * _Q]
    j = pl.program_id(0)

    x = x_ref[...]
    gate = jnp.dot(x[:, 0:_HQ], wg[0][...], preferred_element_type=jnp.float32)
    up = jnp.dot(x[:, 0:_HQ], wu[0][...], preferred_element_type=jnp.float32)
    for q in range(1, _Q):
        xq = x[:, q * _HQ:(q + 1) * _HQ]
        gate = gate + jnp.dot(xq, wg[q][...], preferred_element_type=jnp.float32)
        up = up + jnp.dot(xq, wu[q][...], preferred_element_type=jnp.float32)
    act = gate * jax.nn.sigmoid(gate) * up

    for q in range(_Q):
        contrib = jnp.dot(act, wd[q][...], preferred_element_type=jnp.float32)
        sl = pl.ds(q * _NQ, _NQ)

        @pl.when(j == 0)
        def _init(contrib=contrib, sl=sl):
            o_ref[:, sl] = contrib

        @pl.when(j > 0)
        def _acc(contrib=contrib, sl=sl):
            o_ref[:, sl] += contrib


def kernel(x, W_gate, W_up, W_down):
    wspecs = [pl.BlockSpec((_HQ, _BJ), lambda j, q=q: (q, j))
              for q in range(_Q)]
    dspecs = [pl.BlockSpec((_BJ, _NQ), lambda j, q=q: (j, q))
              for q in range(_Q)]
    return pl.pallas_call(
        _mlp_kernel,
        grid=(_INTER // _BJ,),
        in_specs=([pl.BlockSpec((_TOKENS, _HIDDEN), lambda j: (0, 0))]
                  + wspecs + wspecs + dspecs),
        out_specs=pl.BlockSpec((_TOKENS, _HIDDEN), lambda j: (0, 0)),
        out_shape=jax.ShapeDtypeStruct((_TOKENS, _HIDDEN), jnp.float32),
    )(x, *([W_gate] * _Q), *([W_up] * _Q), *([W_down] * _Q))


# speedup vs baseline: 1.0321x; 1.0321x over previous
import jax
import jax.numpy as jnp
from jax.experimental import pallas as pl

_HIDDEN = 2048
_INTER = 2048
_TOKENS = 32
_BJ = 256
_Q = 4
_QD = 2
_HQ = _HIDDEN // _Q
_DQ = _BJ // _QD


def _mlp_kernel(x_ref, *refs):
    wg = refs[0:_Q]
    wu = refs[_Q:2 * _Q]
    wd = refs[2 * _Q:2 * _Q + _QD]
    o_ref = refs[2 * _Q + _QD]
    j = pl.program_id(0)

    x = x_ref[...]
    gate = jnp.dot(x[:, 0:_HQ], wg[0][...], preferred_element_type=jnp.float32)
    up = jnp.dot(x[:, 0:_HQ], wu[0][...], preferred_element_type=jnp.float32)
    for q in range(1, _Q):
        xq = x[:, q * _HQ:(q + 1) * _HQ]
        gate = gate + jnp.dot(xq, wg[q][...], preferred_element_type=jnp.float32)
        up = up + jnp.dot(xq, wu[q][...], preferred_element_type=jnp.float32)
    act = gate * jax.nn.sigmoid(gate) * up

    contrib = jnp.dot(act[:, 0:_DQ], wd[0][...],
                      preferred_element_type=jnp.float32)
    for q in range(1, _QD):
        contrib = contrib + jnp.dot(act[:, q * _DQ:(q + 1) * _DQ], wd[q][...],
                                    preferred_element_type=jnp.float32)

    @pl.when(j == 0)
    def _init():
        o_ref[...] = contrib

    @pl.when(j > 0)
    def _acc():
        o_ref[...] += contrib


def kernel(x, W_gate, W_up, W_down):
    wspecs = [pl.BlockSpec((_HQ, _BJ), lambda j, q=q: (q, j))
              for q in range(_Q)]
    dspecs = [pl.BlockSpec((_DQ, _HIDDEN), lambda j, q=q: (_QD * j + q, 0))
              for q in range(_QD)]
    return pl.pallas_call(
        _mlp_kernel,
        grid=(_INTER // _BJ,),
        in_specs=([pl.BlockSpec((_TOKENS, _HIDDEN), lambda j: (0, 0))]
                  + wspecs + wspecs + dspecs),
        out_specs=pl.BlockSpec((_TOKENS, _HIDDEN), lambda j: (0, 0)),
        out_shape=jax.ShapeDtypeStruct((_TOKENS, _HIDDEN), jnp.float32),
    )(x, *([W_gate] * _Q), *([W_up] * _Q), *([W_down] * _QD))


# speedup vs baseline: 1.0382x; 1.0058x over previous
import jax
import jax.numpy as jnp
from jax.experimental import pallas as pl

_HIDDEN = 2048
_INTER = 2048
_TOKENS = 32
_BJ = 256
_Q = 8
_QD = 4
_HQ = _HIDDEN // _Q
_DQ = _BJ // _QD


def _mlp_kernel(x_ref, *refs):
    wg = refs[0:_Q]
    wu = refs[_Q:2 * _Q]
    wd = refs[2 * _Q:2 * _Q + _QD]
    o_ref = refs[2 * _Q + _QD]
    j = pl.program_id(0)

    x = x_ref[...]
    gate = jnp.dot(x[:, 0:_HQ], wg[0][...], preferred_element_type=jnp.float32)
    up = jnp.dot(x[:, 0:_HQ], wu[0][...], preferred_element_type=jnp.float32)
    for q in range(1, _Q):
        xq = x[:, q * _HQ:(q + 1) * _HQ]
        gate = gate + jnp.dot(xq, wg[q][...], preferred_element_type=jnp.float32)
        up = up + jnp.dot(xq, wu[q][...], preferred_element_type=jnp.float32)
    act = gate * jax.nn.sigmoid(gate) * up

    contrib = jnp.dot(act[:, 0:_DQ], wd[0][...],
                      preferred_element_type=jnp.float32)
    for q in range(1, _QD):
        contrib = contrib + jnp.dot(act[:, q * _DQ:(q + 1) * _DQ], wd[q][...],
                                    preferred_element_type=jnp.float32)

    @pl.when(j == 0)
    def _init():
        o_ref[...] = contrib

    @pl.when(j > 0)
    def _acc():
        o_ref[...] += contrib


def kernel(x, W_gate, W_up, W_down):
    wspecs = [pl.BlockSpec((_HQ, _BJ), lambda j, q=q: (q, j))
              for q in range(_Q)]
    dspecs = [pl.BlockSpec((_DQ, _HIDDEN), lambda j, q=q: (_QD * j + q, 0))
              for q in range(_QD)]
    return pl.pallas_call(
        _mlp_kernel,
        grid=(_INTER // _BJ,),
        in_specs=([pl.BlockSpec((_TOKENS, _HIDDEN), lambda j: (0, 0))]
                  + wspecs + wspecs + dspecs),
        out_specs=pl.BlockSpec((_TOKENS, _HIDDEN), lambda j: (0, 0)),
        out_shape=jax.ShapeDtypeStruct((_TOKENS, _HIDDEN), jnp.float32),
    )(x, *([W_gate] * _Q), *([W_up] * _Q), *([W_down] * _QD))


# speedup vs baseline: 1.0427x; 1.0044x over previous
import jax
import jax.numpy as jnp
from jax.experimental import pallas as pl

_HIDDEN = 2048
_INTER = 2048
_TOKENS = 32
_BJ = 256
_Q = 4
_QD = 4
_HQ = _HIDDEN // _Q
_DQ = _BJ // _QD


def _mlp_kernel(x_ref, *refs):
    wg = refs[0:_Q]
    wu = refs[_Q:2 * _Q]
    wd = refs[2 * _Q:2 * _Q + _QD]
    o_ref = refs[2 * _Q + _QD]
    j = pl.program_id(0)

    x = x_ref[...]
    gate = jnp.dot(x[:, 0:_HQ], wg[0][...], preferred_element_type=jnp.float32)
    up = jnp.dot(x[:, 0:_HQ], wu[0][...], preferred_element_type=jnp.float32)
    for q in range(1, _Q):
        xq = x[:, q * _HQ:(q + 1) * _HQ]
        gate = gate + jnp.dot(xq, wg[q][...], preferred_element_type=jnp.float32)
        up = up + jnp.dot(xq, wu[q][...], preferred_element_type=jnp.float32)
    act = gate * jax.nn.sigmoid(gate) * up

    contrib = jnp.dot(act[:, 0:_DQ], wd[0][...],
                      preferred_element_type=jnp.float32)
    for q in range(1, _QD):
        contrib = contrib + jnp.dot(act[:, q * _DQ:(q + 1) * _DQ], wd[q][...],
                                    preferred_element_type=jnp.float32)

    @pl.when(j == 0)
    def _init():
        o_ref[...] = contrib

    @pl.when(j > 0)
    def _acc():
        o_ref[...] += contrib


def kernel(x, W_gate, W_up, W_down):
    wspecs = [pl.BlockSpec((_HQ, _BJ), lambda j, q=q: (q, j))
              for q in range(_Q)]
    dspecs = [pl.BlockSpec((_DQ, _HIDDEN), lambda j, q=q: (_QD * j + q, 0))
              for q in range(_QD)]
    return pl.pallas_call(
        _mlp_kernel,
        grid=(_INTER // _BJ,),
        in_specs=([pl.BlockSpec((_TOKENS, _HIDDEN), lambda j: (0, 0))]
                  + wspecs + wspecs + dspecs),
        out_specs=pl.BlockSpec((_TOKENS, _HIDDEN), lambda j: (0, 0)),
        out_shape=jax.ShapeDtypeStruct((_TOKENS, _HIDDEN), jnp.float32),
    )(x, *([W_gate] * _Q), *([W_up] * _Q), *([W_down] * _QD))


# speedup vs baseline: 1.0612x; 1.0178x over previous
import jax
import jax.numpy as jnp
from jax.experimental import pallas as pl

_HIDDEN = 2048
_INTER = 2048
_TOKENS = 32
_BJ = 512
_Q = 4
_QD = 4
_HQ = _HIDDEN // _Q
_DQ = _BJ // _QD


def _mlp_kernel(x_ref, *refs):
    wg = refs[0:_Q]
    wu = refs[_Q:2 * _Q]
    wd = refs[2 * _Q:2 * _Q + _QD]
    o_ref = refs[2 * _Q + _QD]
    j = pl.program_id(0)

    x = x_ref[...]
    gate = jnp.dot(x[:, 0:_HQ], wg[0][...], preferred_element_type=jnp.float32)
    up = jnp.dot(x[:, 0:_HQ], wu[0][...], preferred_element_type=jnp.float32)
    for q in range(1, _Q):
        xq = x[:, q * _HQ:(q + 1) * _HQ]
        gate = gate + jnp.dot(xq, wg[q][...], preferred_element_type=jnp.float32)
        up = up + jnp.dot(xq, wu[q][...], preferred_element_type=jnp.float32)
    act = gate * jax.nn.sigmoid(gate) * up

    contrib = jnp.dot(act[:, 0:_DQ], wd[0][...],
                      preferred_element_type=jnp.float32)
    for q in range(1, _QD):
        contrib = contrib + jnp.dot(act[:, q * _DQ:(q + 1) * _DQ], wd[q][...],
                                    preferred_element_type=jnp.float32)

    @pl.when(j == 0)
    def _init():
        o_ref[...] = contrib

    @pl.when(j > 0)
    def _acc():
        o_ref[...] += contrib


def kernel(x, W_gate, W_up, W_down):
    wspecs = [pl.BlockSpec((_HQ, _BJ), lambda j, q=q: (q, j))
              for q in range(_Q)]
    dspecs = [pl.BlockSpec((_DQ, _HIDDEN), lambda j, q=q: (_QD * j + q, 0))
              for q in range(_QD)]
    return pl.pallas_call(
        _mlp_kernel,
        grid=(_INTER // _BJ,),
        in_specs=([pl.BlockSpec((_TOKENS, _HIDDEN), lambda j: (0, 0))]
                  + wspecs + wspecs + dspecs),
        out_specs=pl.BlockSpec((_TOKENS, _HIDDEN), lambda j: (0, 0)),
        out_shape=jax.ShapeDtypeStruct((_TOKENS, _HIDDEN), jnp.float32),
    )(x, *([W_gate] * _Q), *([W_up] * _Q), *([W_down] * _QD))
